# baseline (speedup 1.0000x reference)
import functools

import jax
import jax.numpy as jnp
from jax import lax
from jax.experimental import pallas as pl
from jax.experimental.pallas import tpu as pltpu
from jax.experimental.pallas import tpu_sc as plsc

_EPS = 1e-7
_MAXNORM = 1.0 - 4e-3

_N = 10000
_D = 128
_NC = 2
_NS = 16
_EB = 16
_NSLOT = 12
_ROWS_PER_TILE = 632
_N_PAD = _NS * _ROWS_PER_TILE
_TRASH = _N


def _norm(x):
    return jnp.sqrt(jnp.clip(jnp.sum(x * x, axis=-1, keepdims=True), 1e-15, None))


def _project(x):
    n = _norm(x)
    return jnp.where(n > _MAXNORM, x / n * _MAXNORM, x)


def _artanh(z):
    z = jnp.clip(z, -1.0 + _EPS, 1.0 - _EPS)
    return 0.5 * jnp.log((1.0 + z) / (1.0 - z))


def _expmap0(u):
    n = _norm(u)
    return _project(jnp.tanh(n) * u / n)


def _logmap0(y):
    n = _norm(y)
    return _artanh(n) * y / n


def _mobius_matvec(W, x):
    mv = lax.dot_general(x, W, (((1,), (1,)), ((), ())),
                         preferred_element_type=jnp.float32)
    x_n = _norm(x)
    mv_n = _norm(mv)
    res = jnp.tanh(mv_n / x_n * _artanh(x_n)) * mv / mv_n
    return _project(res)


def _mobius_add(x, y):
    x2 = jnp.sum(x * x, -1, keepdims=True)
    y2 = jnp.sum(y * y, -1, keepdims=True)
    xy = jnp.sum(x * y, -1, keepdims=True)
    num = (1.0 + 2.0 * xy + y2) * x + (1.0 - x2) * y
    den = 1.0 + 2.0 * xy + x2 * y2
    return _project(num / jnp.clip(den, 1e-15, None))


def _kappa_linear(x, W, b):
    return _mobius_add(_mobius_matvec(W, x), _expmap0(b))


def _dis_from_degp(degp):
    deg = jnp.max(degp[0] + degp[1], axis=-1, keepdims=True) + 1.0
    return lax.rsqrt(deg)


def _dense1_body(x_ref, w_ref, b_ref, degp_ref, y_ref):
    dis = _dis_from_degp(degp_ref[...])
    xm = _expmap0(x_ref[...])
    h = _kappa_linear(xm, w_ref[...], b_ref[...])
    y_ref[...] = dis * _logmap0(h)


def _dense2_body(accp_ref, y1_ref, degp_ref, w_ref, b_ref, y2_ref):
    dis = _dis_from_degp(degp_ref[...])
    agg = accp_ref[0] + accp_ref[1] + y1_ref[...]
    out1 = _expmap0(dis * agg)
    h = _kappa_linear(out1, w_ref[...], b_ref[...])
    y2_ref[...] = dis * _logmap0(h)


def _final_body(accp_ref, y2_ref, degp_ref, out_ref):
    dis = _dis_from_degp(degp_ref[...])
    agg = accp_ref[0] + accp_ref[1] + y2_ref[...]
    out_ref[...] = _expmap0(dis * agg)


_RB = 1000


def _row_spec(i):
    return (i, 0)


def _fixed_spec(i):
    return (0, 0)


def _part_spec(i):
    return (0, i, 0)


def _dense1(x, W, b2d, degp):
    return pl.pallas_call(
        _dense1_body,
        grid=(_N // _RB,),
        in_specs=[
            pl.BlockSpec((_RB, _D), _row_spec),
            pl.BlockSpec((_D, _D), _fixed_spec),
            pl.BlockSpec((1, _D), _fixed_spec),
            pl.BlockSpec((_NC, _RB, _D), _part_spec),
        ],
        out_specs=pl.BlockSpec((_RB, _D), _row_spec),
        out_shape=jax.ShapeDtypeStruct((_N, _D), jnp.float32),
    )(x, W, b2d, degp)


def _dense2(accp, y1, degp, W, b2d):
    return pl.pallas_call(
        _dense2_body,
        grid=(_N // _RB,),
        in_specs=[
            pl.BlockSpec((_NC, _RB, _D), _part_spec),
            pl.BlockSpec((_RB, _D), _row_spec),
            pl.BlockSpec((_NC, _RB, _D), _part_spec),
            pl.BlockSpec((_D, _D), _fixed_spec),
            pl.BlockSpec((1, _D), _fixed_spec),
        ],
        out_specs=pl.BlockSpec((_RB, _D), _row_spec),
        out_shape=jax.ShapeDtypeStruct((_N, _D), jnp.float32),
    )(accp, y1, degp, W, b2d)


def _final(accp, y2, degp):
    return pl.pallas_call(
        _final_body,
        grid=(_N // _RB,),
        in_specs=[
            pl.BlockSpec((_NC, _RB, _D), _part_spec),
            pl.BlockSpec((_RB, _D), _row_spec),
            pl.BlockSpec((_NC, _RB, _D), _part_spec),
        ],
        out_specs=pl.BlockSpec((_RB, _D), _row_spec),
        out_shape=jax.ShapeDtypeStruct((_N, _D), jnp.float32),
    )(accp, y2, degp)


def _edge_layout(n_edges_padded):
    per_tile = n_edges_padded // (_NC * _NS)
    return per_tile // _EB


def _idx(a, b):
    return a.at[pl.ds(b * _EB, _EB)]


def _deg_body(nblk, colp_hbm, zdeg_hbm, ones_hbm, degp_hbm,
              cidx_a, ones_v, ssem0, ssem1, deg_sh):
    c = lax.axis_index("c")
    s = lax.axis_index("s")
    wid = s * _NC + c
    pltpu.sync_copy(ones_hbm, ones_v)
    pltpu.sync_copy(zdeg_hbm, deg_sh.at[pl.ds(s * _ROWS_PER_TILE, _ROWS_PER_TILE)])
    pltpu.sync_copy(colp_hbm.at[wid], cidx_a)
    plsc.subcore_barrier()

    ssem = (ssem0, ssem1)

    def s_wait(b, slot):
        pltpu.make_async_copy(ones_v, deg_sh.at[_idx(cidx_a, b)], ssem[slot]).wait()

    def fire(b, slot):
        pltpu.async_copy(ones_v, deg_sh.at[_idx(cidx_a, b)], ssem[slot], add=True)

    fire(0, 0)
    fire(1, 1)

    def pair(i, carry):
        b = 2 + 2 * i
        s_wait(b - 2, 0)
        fire(b, 0)
        s_wait(b - 1, 1)
        fire(b + 1, 1)
        return carry

    lax.fori_loop(0, (nblk - 3) // 2, pair, 0)
    s_wait(nblk - 3, 0)
    fire(nblk - 1, 0)
    s_wait(nblk - 2, 1)
    s_wait(nblk - 1, 0)
    plsc.subcore_barrier()
    pltpu.sync_copy(deg_sh.at[pl.ds(s * _ROWS_PER_TILE, _ROWS_PER_TILE)],
                    degp_hbm.at[c, pl.ds(s * _ROWS_PER_TILE, _ROWS_PER_TILE)])


def _scat_body(nblk, rowp_hbm, colp_hbm, y_hbm, zacc_hbm, accp_hbm, *scr):
    ridx_a, cidx_a = scr[0], scr[1]
    rows = scr[2:2 + _NSLOT]
    gsem = scr[2 + _NSLOT:2 + 2 * _NSLOT]
    ssem = scr[2 + 2 * _NSLOT:2 + 3 * _NSLOT]
    acc_sh = scr[-1]
    ns = _NSLOT
    c = lax.axis_index("c")
    s = lax.axis_index("s")
    wid = s * _NC + c
    pltpu.sync_copy(zacc_hbm, acc_sh.at[pl.ds(s * _ROWS_PER_TILE, _ROWS_PER_TILE)])
    pltpu.sync_copy(rowp_hbm.at[wid], ridx_a)
    pltpu.sync_copy(colp_hbm.at[wid], cidx_a)
    plsc.subcore_barrier()

    def g_copy(b, slot):
        return pltpu.make_async_copy(y_hbm.at[_idx(ridx_a, b)], rows[slot], gsem[slot])

    def s_wait(b, slot):
        pltpu.make_async_copy(rows[slot], acc_sh.at[_idx(cidx_a, b)], ssem[slot]).wait()

    def scat(b, slot):
        pltpu.async_copy(rows[slot], acc_sh.at[_idx(cidx_a, b)], ssem[slot], add=True)

    for b0 in range(ns):
        g_copy(b0, b0).start()
    g_copy(0, 0).wait()
    scat(0, 0)

    def body(i, carry):
        b = 1 + ns * i
        for j in range(ns):
            sl = (1 + j) % ns
            s_wait(b + j - 1, j % ns)
            g_copy(b + j + ns - 1, j % ns).start()
            g_copy(b + j, sl).wait()
            scat(b + j, sl)
        return carry

    lax.fori_loop(0, (nblk - ns - 1) // ns, body, 0)
    b = nblk - ns
    s_wait(b - 1, 0)
    g_copy(nblk - 1, 0).start()
    for j in range(ns):
        sl = (1 + j) % ns
        if j > 0:
            s_wait(b + j - 1, j % ns)
        g_copy(b + j, sl).wait()
        scat(b + j, sl)
    s_wait(nblk - 1, (nblk - 1) % ns)
    plsc.subcore_barrier()
    pltpu.sync_copy(acc_sh.at[pl.ds(s * _ROWS_PER_TILE, _ROWS_PER_TILE)],
                    accp_hbm.at[c, pl.ds(s * _ROWS_PER_TILE, _ROWS_PER_TILE)])


def _sc_mesh():
    return plsc.VectorSubcoreMesh(core_axis_name="c", subcore_axis_name="s")


def _sc_degree(colp, zdeg, ones, nblk):
    return pl.kernel(
        functools.partial(_deg_body, nblk),
        out_type=jax.ShapeDtypeStruct((_NC, _N_PAD, _D), jnp.float32),
        mesh=_sc_mesh(),
        scratch_types=[
            pltpu.VMEM((nblk * _EB,), jnp.int32),
            pltpu.VMEM((_EB, _D), jnp.float32),
            pltpu.SemaphoreType.DMA,
            pltpu.SemaphoreType.DMA,
            pltpu.VMEM_SHARED((_N_PAD, _D), jnp.float32),
        ],
    )(colp, zdeg, ones)


def _sc_scatter(rowp, colp, y, zacc, nblk):
    return pl.kernel(
        functools.partial(_scat_body, nblk),
        out_type=jax.ShapeDtypeStruct((_NC, _N_PAD, _D), jnp.float32),
        mesh=_sc_mesh(),
        scratch_types=(
            [pltpu.VMEM((nblk * _EB,), jnp.int32)] * 2
            + [pltpu.VMEM((_EB, _D), jnp.float32)] * _NSLOT
            + [pltpu.SemaphoreType.DMA] * (2 * _NSLOT)
            + [pltpu.VMEM_SHARED((_N_PAD, _D), jnp.float32)]
        ),
    )(rowp, colp, y, zacc)


def kernel(x, edge_index, W1, b1, W2, b2):
    n, d = x.shape
    assert n == _N and d == _D
    ei = edge_index.astype(jnp.int32)
    row, col = ei[0], ei[1]
    e = row.shape[0]
    chunk = _NC * _NS * _EB
    e_pad = ((e + chunk - 1) // chunk) * chunk
    pad = e_pad - e
    nblk = _edge_layout(e_pad)
    assert nblk >= 2 * _NSLOT + 1 and nblk % 2 == 1 and nblk % _NSLOT == 1
    nw = _NC * _NS
    rowp = jnp.concatenate([row, jnp.zeros((pad,), jnp.int32)]).reshape(nw, nblk * _EB)
    colp = jnp.concatenate([col, jnp.full((pad,), _TRASH, jnp.int32)]).reshape(nw, nblk * _EB)

    zacc = jnp.zeros((_ROWS_PER_TILE, _D), jnp.float32)
    ones = jnp.ones((_EB, _D), jnp.float32)
    b1_2d = b1.reshape(1, _D).astype(jnp.float32)
    b2_2d = b2.reshape(1, _D).astype(jnp.float32)

    degp = _sc_degree(colp, zacc, ones, nblk)
    y1 = _dense1(x, W1, b1_2d, degp)
    accp1 = _sc_scatter(rowp, colp, y1, zacc, nblk)
    y2 = _dense2(accp1, y1, degp, W2, b2_2d)
    accp2 = _sc_scatter(rowp, colp, y2, zacc, nblk)
    return _final(accp2, y2, degp)


# speedup vs baseline: 1.0432x; 1.0432x over previous
import functools

import jax
import jax.numpy as jnp
from jax import lax
from jax.experimental import pallas as pl
from jax.experimental.pallas import tpu as pltpu
from jax.experimental.pallas import tpu_sc as plsc

_EPS = 1e-7
_MAXNORM = 1.0 - 4e-3

_N = 10000
_D = 128
_NC = 2
_NS = 16
_EB = 24
_NSLOT = 8
_ROWS_PER_TILE = 632
_N_PAD = _NS * _ROWS_PER_TILE
_TRASH = _N


def _norm(x):
    return jnp.sqrt(jnp.clip(jnp.sum(x * x, axis=-1, keepdims=True), 1e-15, None))


def _project(x):
    n = _norm(x)
    return jnp.where(n > _MAXNORM, x / n * _MAXNORM, x)


def _artanh(z):
    z = jnp.clip(z, -1.0 + _EPS, 1.0 - _EPS)
    return 0.5 * jnp.log((1.0 + z) / (1.0 - z))


def _expmap0(u):
    n = _norm(u)
    return _project(jnp.tanh(n) * u / n)


def _logmap0(y):
    n = _norm(y)
    return _artanh(n) * y / n


def _mobius_matvec(W, x):
    mv = lax.dot_general(x, W, (((1,), (1,)), ((), ())),
                         preferred_element_type=jnp.float32)
    x_n = _norm(x)
    mv_n = _norm(mv)
    res = jnp.tanh(mv_n / x_n * _artanh(x_n)) * mv / mv_n
    return _project(res)


def _mobius_add(x, y):
    x2 = jnp.sum(x * x, -1, keepdims=True)
    y2 = jnp.sum(y * y, -1, keepdims=True)
    xy = jnp.sum(x * y, -1, keepdims=True)
    num = (1.0 + 2.0 * xy + y2) * x + (1.0 - x2) * y
    den = 1.0 + 2.0 * xy + x2 * y2
    return _project(num / jnp.clip(den, 1e-15, None))


def _kappa_linear(x, W, b):
    return _mobius_add(_mobius_matvec(W, x), _expmap0(b))


def _dis_from_degp(degp):
    deg = jnp.max(degp[0] + degp[1], axis=-1, keepdims=True) + 1.0
    return lax.rsqrt(deg)


def _dense1_body(x_ref, w_ref, b_ref, degp_ref, y_ref):
    dis = _dis_from_degp(degp_ref[...])
    xm = _expmap0(x_ref[...])
    h = _kappa_linear(xm, w_ref[...], b_ref[...])
    y_ref[...] = dis * _logmap0(h)


def _dense2_body(accp_ref, y1_ref, degp_ref, w_ref, b_ref, y2_ref):
    dis = _dis_from_degp(degp_ref[...])
    agg = accp_ref[0] + accp_ref[1] + y1_ref[...]
    out1 = _expmap0(dis * agg)
    h = _kappa_linear(out1, w_ref[...], b_ref[...])
    y2_ref[...] = dis * _logmap0(h)


def _final_body(accp_ref, y2_ref, degp_ref, out_ref):
    dis = _dis_from_degp(degp_ref[...])
    agg = accp_ref[0] + accp_ref[1] + y2_ref[...]
    out_ref[...] = _expmap0(dis * agg)


_RB = 1000


def _row_spec(i):
    return (i, 0)


def _fixed_spec(i):
    return (0, 0)


def _part_spec(i):
    return (0, i, 0)


def _dense1(x, W, b2d, degp):
    return pl.pallas_call(
        _dense1_body,
        grid=(_N // _RB,),
        in_specs=[
            pl.BlockSpec((_RB, _D), _row_spec),
            pl.BlockSpec((_D, _D), _fixed_spec),
            pl.BlockSpec((1, _D), _fixed_spec),
            pl.BlockSpec((_NC, _RB, _D), _part_spec),
        ],
        out_specs=pl.BlockSpec((_RB, _D), _row_spec),
        out_shape=jax.ShapeDtypeStruct((_N, _D), jnp.float32),
    )(x, W, b2d, degp)


def _dense2(accp, y1, degp, W, b2d):
    return pl.pallas_call(
        _dense2_body,
        grid=(_N // _RB,),
        in_specs=[
            pl.BlockSpec((_NC, _RB, _D), _part_spec),
            pl.BlockSpec((_RB, _D), _row_spec),
            pl.BlockSpec((_NC, _RB, _D), _part_spec),
            pl.BlockSpec((_D, _D), _fixed_spec),
            pl.BlockSpec((1, _D), _fixed_spec),
        ],
        out_specs=pl.BlockSpec((_RB, _D), _row_spec),
        out_shape=jax.ShapeDtypeStruct((_N, _D), jnp.float32),
    )(accp, y1, degp, W, b2d)


def _final(accp, y2, degp):
    return pl.pallas_call(
        _final_body,
        grid=(_N // _RB,),
        in_specs=[
            pl.BlockSpec((_NC, _RB, _D), _part_spec),
            pl.BlockSpec((_RB, _D), _row_spec),
            pl.BlockSpec((_NC, _RB, _D), _part_spec),
        ],
        out_specs=pl.BlockSpec((_RB, _D), _row_spec),
        out_shape=jax.ShapeDtypeStruct((_N, _D), jnp.float32),
    )(accp, y2, degp)


def _edge_layout(n_edges_padded):
    per_tile = n_edges_padded // (_NC * _NS)
    return per_tile // _EB


def _idx(a, b):
    return a.at[pl.ds(b * _EB, _EB)]


def _deg_body(nblk, colp_hbm, zdeg_hbm, ones_hbm, degp_hbm, *scr):
    cidx_a, ones_v = scr[0], scr[1]
    ssem = scr[2:2 + _NSLOT]
    deg_sh = scr[-1]
    ns = _NSLOT
    c = lax.axis_index("c")
    s = lax.axis_index("s")
    wid = s * _NC + c
    pltpu.sync_copy(ones_hbm, ones_v)
    pltpu.sync_copy(zdeg_hbm, deg_sh.at[pl.ds(s * _ROWS_PER_TILE, _ROWS_PER_TILE)])
    pltpu.sync_copy(colp_hbm.at[wid], cidx_a)
    plsc.subcore_barrier()

    def s_wait(b, slot):
        pltpu.make_async_copy(ones_v, deg_sh.at[_idx(cidx_a, b)], ssem[slot]).wait()

    def fire(b, slot):
        pltpu.async_copy(ones_v, deg_sh.at[_idx(cidx_a, b)], ssem[slot], add=True)

    for b0 in range(ns):
        fire(b0, b0)

    def body(i, carry):
        b = ns + ns * i
        for j in range(ns):
            s_wait(b + j - ns, j)
            fire(b + j, j)
        return carry

    lax.fori_loop(0, (nblk - ns - 1) // ns, body, 0)
    s_wait(nblk - 1 - ns, (nblk - 1) % ns)
    fire(nblk - 1, (nblk - 1) % ns)
    for b0 in range(nblk - ns, nblk):
        s_wait(b0, b0 % ns)
    plsc.subcore_barrier()
    pltpu.sync_copy(deg_sh.at[pl.ds(s * _ROWS_PER_TILE, _ROWS_PER_TILE)],
                    degp_hbm.at[c, pl.ds(s * _ROWS_PER_TILE, _ROWS_PER_TILE)])


def _scat_body(nblk, rowp_hbm, colp_hbm, y_hbm, zacc_hbm, accp_hbm, *scr):
    ridx_a, cidx_a = scr[0], scr[1]
    rows = scr[2:2 + _NSLOT]
    gsem = scr[2 + _NSLOT:2 + 2 * _NSLOT]
    ssem = scr[2 + 2 * _NSLOT:2 + 3 * _NSLOT]
    acc_sh = scr[-1]
    ns = _NSLOT
    c = lax.axis_index("c")
    s = lax.axis_index("s")
    wid = s * _NC + c
    pltpu.sync_copy(zacc_hbm, acc_sh.at[pl.ds(s * _ROWS_PER_TILE, _ROWS_PER_TILE)])
    pltpu.sync_copy(rowp_hbm.at[wid], ridx_a)
    pltpu.sync_copy(colp_hbm.at[wid], cidx_a)
    plsc.subcore_barrier()

    def g_copy(b, slot):
        return pltpu.make_async_copy(y_hbm.at[_idx(ridx_a, b)], rows[slot], gsem[slot])

    def s_wait(b, slot):
        pltpu.make_async_copy(rows[slot], acc_sh.at[_idx(cidx_a, b)], ssem[slot]).wait()

    def scat(b, slot):
        pltpu.async_copy(rows[slot], acc_sh.at[_idx(cidx_a, b)], ssem[slot], add=True)

    for b0 in range(ns):
        g_copy(b0, b0).start()
    g_copy(0, 0).wait()
    scat(0, 0)

    def body(i, carry):
        b = 1 + ns * i
        for j in range(ns):
            sl = (1 + j) % ns
            s_wait(b + j - 1, j % ns)
            g_copy(b + j + ns - 1, j % ns).start()
            g_copy(b + j, sl).wait()
            scat(b + j, sl)
        return carry

    lax.fori_loop(0, (nblk - ns - 1) // ns, body, 0)
    b = nblk - ns
    s_wait(b - 1, 0)
    g_copy(nblk - 1, 0).start()
    for j in range(ns):
        sl = (1 + j) % ns
        if j > 0:
            s_wait(b + j - 1, j % ns)
        g_copy(b + j, sl).wait()
        scat(b + j, sl)
    s_wait(nblk - 1, (nblk - 1) % ns)
    plsc.subcore_barrier()
    pltpu.sync_copy(acc_sh.at[pl.ds(s * _ROWS_PER_TILE, _ROWS_PER_TILE)],
                    accp_hbm.at[c, pl.ds(s * _ROWS_PER_TILE, _ROWS_PER_TILE)])


def _sc_mesh():
    return plsc.VectorSubcoreMesh(core_axis_name="c", subcore_axis_name="s")


def _sc_degree(colp, zdeg, ones, nblk):
    return pl.kernel(
        functools.partial(_deg_body, nblk),
        out_type=jax.ShapeDtypeStruct((_NC, _N_PAD, _D), jnp.float32),
        mesh=_sc_mesh(),
        scratch_types=(
            [pltpu.VMEM((nblk * _EB,), jnp.int32),
             pltpu.VMEM((_EB, _D), jnp.float32)]
            + [pltpu.SemaphoreType.DMA] * _NSLOT
            + [pltpu.VMEM_SHARED((_N_PAD, _D), jnp.float32)]
        ),
    )(colp, zdeg, ones)


def _sc_scatter(rowp, colp, y, zacc, nblk):
    return pl.kernel(
        functools.partial(_scat_body, nblk),
        out_type=jax.ShapeDtypeStruct((_NC, _N_PAD, _D), jnp.float32),
        mesh=_sc_mesh(),
        scratch_types=(
            [pltpu.VMEM((nblk * _EB,), jnp.int32)] * 2
            + [pltpu.VMEM((_EB, _D), jnp.float32)] * _NSLOT
            + [pltpu.SemaphoreType.DMA] * (2 * _NSLOT)
            + [pltpu.VMEM_SHARED((_N_PAD, _D), jnp.float32)]
        ),
    )(rowp, colp, y, zacc)


def kernel(x, edge_index, W1, b1, W2, b2):
    n, d = x.shape
    assert n == _N and d == _D
    ei = edge_index.astype(jnp.int32)
    row, col = ei[0], ei[1]
    e = row.shape[0]
    chunk = _NC * _NS * _EB
    e_pad = ((e + chunk - 1) // chunk) * chunk
    pad = e_pad - e
    nblk = _edge_layout(e_pad)
    assert nblk >= 2 * _NSLOT + 1 and nblk % 2 == 1 and nblk % _NSLOT == 1
    nw = _NC * _NS
    rowp = jnp.concatenate([row, jnp.zeros((pad,), jnp.int32)]).reshape(nw, nblk * _EB)
    colp = jnp.concatenate([col, jnp.full((pad,), _TRASH, jnp.int32)]).reshape(nw, nblk * _EB)

    zacc = jnp.zeros((_ROWS_PER_TILE, _D), jnp.float32)
    ones = jnp.ones((_EB, _D), jnp.float32)
    b1_2d = b1.reshape(1, _D).astype(jnp.float32)
    b2_2d = b2.reshape(1, _D).astype(jnp.float32)

    degp = _sc_degree(colp, zacc, ones, nblk)
    y1 = _dense1(x, W1, b1_2d, degp)
    accp1 = _sc_scatter(rowp, colp, y1, zacc, nblk)
    y2 = _dense2(accp1, y1, degp, W2, b2_2d)
    accp2 = _sc_scatter(rowp, colp, y2, zacc, nblk)
    return _final(accp2, y2, degp)
